# baseline (speedup 1.0000x reference)
import functools

import jax
import jax.numpy as jnp
from jax import lax
from jax.experimental import pallas as pl
from jax.experimental.pallas import tpu as pltpu
from jax.experimental.pallas import tpu_sc as plsc

_B = 4096
_D = 64
_V = 100000


def _make_sc_gather():
    info = plsc.get_sparse_core_info()
    nw = info.num_cores * info.num_subcores
    b_per_w = _B // nw
    mesh = plsc.VectorSubcoreMesh(core_axis_name="c", subcore_axis_name="s")

    @functools.partial(
        pl.kernel,
        mesh=mesh,
        out_type=jax.ShapeDtypeStruct((_B, _D), jnp.float32),
        scratch_types=[
            pltpu.VMEM((b_per_w,), jnp.int32),
            pltpu.VMEM((b_per_w, _D), jnp.float32),
            pltpu.SemaphoreType.DMA,
        ],
        compiler_params=pltpu.CompilerParams(use_tc_tiling_on_sc=False),
    )
    def gather_kernel(table_hbm, idx_hbm, out_hbm, idx_v, rows_v, sem):
        wid = lax.axis_index("s") * info.num_cores + lax.axis_index("c")
        base = wid * b_per_w
        pltpu.sync_copy(idx_hbm.at[pl.ds(base, b_per_w)], idx_v)
        pltpu.async_copy(table_hbm.at[idx_v], rows_v, sem).wait()
        pltpu.sync_copy(rows_v, out_hbm.at[pl.ds(base, b_per_w)])

    return gather_kernel


def _mm_body(h_ref, w2_ref, o_ref):
    o_ref[...] = lax.dot_general(
        h_ref[...],
        w2_ref[...],
        (((1,), (1,)), ((), ())),
        preferred_element_type=jnp.float32,
    )


def _projection(hidden, W2, tb, tn):
    nv = pl.cdiv(_V, tn)
    nb = _B // tb
    return pl.pallas_call(
        _mm_body,
        grid=(nv, nb),
        in_specs=[
            pl.BlockSpec((tb, _D), lambda v, b: (b, 0)),
            pl.BlockSpec((tn, _D), lambda v, b: (v, 0)),
        ],
        out_specs=pl.BlockSpec((tb, tn), lambda v, b: (b, v)),
        out_shape=jax.ShapeDtypeStruct((_B, _V), jnp.float32),
    )(hidden, W2)


_sc_gather = _make_sc_gather()


@jax.jit
def kernel(X, W, W2):
    hidden = _sc_gather(W, X.astype(jnp.int32))
    return _projection(hidden, W2, 1024, 2048)


# speedup vs baseline: 3.3693x; 3.3693x over previous
import functools

import jax
import jax.numpy as jnp
from jax import lax
from jax.experimental import pallas as pl
from jax.experimental.pallas import tpu as pltpu
from jax.experimental.pallas import tpu_sc as plsc

_B = 4096
_D = 64
_V = 100000


def _make_sc_gather():
    info = plsc.get_sparse_core_info()
    nw = info.num_cores * info.num_subcores
    b_per_w = _B // nw
    mesh = plsc.VectorSubcoreMesh(core_axis_name="c", subcore_axis_name="s")

    @functools.partial(
        pl.kernel,
        mesh=mesh,
        out_type=jax.ShapeDtypeStruct((_B, _D), jnp.float32),
        scratch_types=[
            pltpu.VMEM((b_per_w,), jnp.int32),
            pltpu.VMEM((b_per_w, _D), jnp.float32),
            pltpu.SemaphoreType.DMA,
        ],
        compiler_params=pltpu.CompilerParams(use_tc_tiling_on_sc=False),
    )
    def gather_kernel(table_hbm, idx_hbm, out_hbm, idx_v, rows_v, sem):
        wid = lax.axis_index("s") * info.num_cores + lax.axis_index("c")
        base = wid * b_per_w
        pltpu.sync_copy(idx_hbm.at[pl.ds(base, b_per_w)], idx_v)
        pltpu.async_copy(table_hbm.at[idx_v], rows_v, sem).wait()
        pltpu.sync_copy(rows_v, out_hbm.at[pl.ds(base, b_per_w)])

    return gather_kernel


def _mm_body(h_ref, w2_ref, o_ref):
    o_ref[...] = lax.dot_general(
        w2_ref[...],
        h_ref[...],
        (((1,), (1,)), ((), ())),
        preferred_element_type=jnp.float32,
    )


def _projection_t(hidden, W2, tn):
    nv = pl.cdiv(_V, tn)
    return pl.pallas_call(
        _mm_body,
        grid=(nv,),
        in_specs=[
            pl.BlockSpec((_B, _D), lambda v: (0, 0)),
            pl.BlockSpec((tn, _D), lambda v: (v, 0)),
        ],
        out_specs=pl.BlockSpec((tn, _B), lambda v: (v, 0)),
        out_shape=jax.ShapeDtypeStruct((_V, _B), jnp.float32),
    )(hidden, W2)


_sc_gather = _make_sc_gather()


@jax.jit
def kernel(X, W, W2):
    hidden = _sc_gather(W, X.astype(jnp.int32))
    return _projection_t(hidden, W2, 512).T


# speedup vs baseline: 3.5733x; 1.0606x over previous
import functools

import jax
import jax.numpy as jnp
from jax import lax
from jax.experimental import pallas as pl
from jax.experimental.pallas import tpu as pltpu
from jax.experimental.pallas import tpu_sc as plsc

_B = 4096
_D = 64
_V = 100000


def _make_sc_gather():
    info = plsc.get_sparse_core_info()
    nw = info.num_cores * info.num_subcores
    b_per_w = _B // nw
    mesh = plsc.VectorSubcoreMesh(core_axis_name="c", subcore_axis_name="s")

    @functools.partial(
        pl.kernel,
        mesh=mesh,
        out_type=jax.ShapeDtypeStruct((_B, _D), jnp.float32),
        scratch_types=[
            pltpu.VMEM((b_per_w,), jnp.int32),
            pltpu.VMEM((b_per_w, _D), jnp.float32),
            pltpu.SemaphoreType.DMA,
        ],
        compiler_params=pltpu.CompilerParams(use_tc_tiling_on_sc=False),
    )
    def gather_kernel(table_hbm, idx_hbm, out_hbm, idx_v, rows_v, sem):
        wid = lax.axis_index("s") * info.num_cores + lax.axis_index("c")
        base = wid * b_per_w
        pltpu.sync_copy(idx_hbm.at[pl.ds(base, b_per_w)], idx_v)
        pltpu.async_copy(table_hbm.at[idx_v], rows_v, sem).wait()
        pltpu.sync_copy(rows_v, out_hbm.at[pl.ds(base, b_per_w)])

    return gather_kernel


def _mm_body(h_ref, w2t_ref, o_ref):
    o_ref[...] = lax.dot_general(
        w2t_ref[...],
        h_ref[...],
        (((0,), (1,)), ((), ())),
        preferred_element_type=jnp.float32,
    )


def _projection_t(hidden, W2t, tn):
    nv = pl.cdiv(_V, tn)
    return pl.pallas_call(
        _mm_body,
        grid=(nv,),
        in_specs=[
            pl.BlockSpec((_B, _D), lambda v: (0, 0)),
            pl.BlockSpec((_D, tn), lambda v: (0, v)),
        ],
        out_specs=pl.BlockSpec((tn, _B), lambda v: (v, 0)),
        out_shape=jax.ShapeDtypeStruct((_V, _B), jnp.float32),
    )(hidden, W2t)


_sc_gather = _make_sc_gather()


@jax.jit
def kernel(X, W, W2):
    hidden = _sc_gather(W, X.astype(jnp.int32))
    return _projection_t(hidden, W2.T, 512).T


# speedup vs baseline: 3.6029x; 1.0083x over previous
import functools

import jax
import jax.numpy as jnp
from jax import lax
from jax.experimental import pallas as pl
from jax.experimental.pallas import tpu as pltpu
from jax.experimental.pallas import tpu_sc as plsc

_B = 4096
_D = 64
_V = 100000


def _make_sc_gather():
    info = plsc.get_sparse_core_info()
    nw = info.num_cores * info.num_subcores
    b_per_w = _B // nw
    mesh = plsc.VectorSubcoreMesh(core_axis_name="c", subcore_axis_name="s")

    @functools.partial(
        pl.kernel,
        mesh=mesh,
        out_type=jax.ShapeDtypeStruct((_B, _D), jnp.float32),
        scratch_types=[
            pltpu.VMEM((b_per_w,), jnp.int32),
            pltpu.VMEM((b_per_w, _D), jnp.float32),
            pltpu.SemaphoreType.DMA,
        ],
        compiler_params=pltpu.CompilerParams(use_tc_tiling_on_sc=False),
    )
    def gather_kernel(table_hbm, idx_hbm, out_hbm, idx_v, rows_v, sem):
        wid = lax.axis_index("s") * info.num_cores + lax.axis_index("c")
        base = wid * b_per_w
        pltpu.sync_copy(idx_hbm.at[pl.ds(base, b_per_w)], idx_v)
        pltpu.async_copy(table_hbm.at[idx_v], rows_v, sem).wait()
        pltpu.sync_copy(rows_v, out_hbm.at[pl.ds(base, b_per_w)])

    return gather_kernel


def _mm_body(h_ref, w2t_ref, o_ref):
    o_ref[...] = lax.dot_general(
        w2t_ref[...],
        h_ref[...],
        (((0,), (1,)), ((), ())),
        preferred_element_type=jnp.float32,
    )


def _projection_t(hidden, W2t, tn):
    nv = pl.cdiv(_V, tn)
    return pl.pallas_call(
        _mm_body,
        grid=(nv,),
        in_specs=[
            pl.BlockSpec((_B, _D), lambda v: (0, 0)),
            pl.BlockSpec((_D, tn), lambda v: (0, v)),
        ],
        out_specs=pl.BlockSpec((tn, _B), lambda v: (v, 0)),
        out_shape=jax.ShapeDtypeStruct((_V, _B), jnp.float32),
    )(hidden, W2t)


_sc_gather = _make_sc_gather()


@jax.jit
def kernel(X, W, W2):
    hidden = _sc_gather(W, X.astype(jnp.int32))
    return _projection_t(hidden, W2.T, 1024).T
